# baseline (speedup 1.0000x reference)
import functools

import jax
import jax.numpy as jnp
from jax import lax
from jax.experimental import pallas as pl
from jax.experimental.pallas import tpu as pltpu
from jax.experimental.pallas import tpu_sc as plsc

_LANES = 16


def _format_table(tab_t, num_cores, num_workers):
    d, v = tab_t.shape
    full_tiles = v // 128
    rem = v - full_tiles * 128
    upw = -(-full_tiles // num_workers)

    mesh = plsc.VectorSubcoreMesh(core_axis_name="c", subcore_axis_name="s")

    @functools.partial(
        pl.kernel,
        mesh=mesh,
        out_type=jax.ShapeDtypeStruct((v * d,), jnp.float32),
        scratch_types=[
            pltpu.VMEM((d, 129), jnp.float32),
            pltpu.VMEM((d, 129), jnp.float32),
            pltpu.VMEM((128 * d,), jnp.float32),
            pltpu.VMEM((128 * d,), jnp.float32),
            pltpu.SemaphoreType.DMA((2,)),
            pltpu.SemaphoreType.DMA((2,)),
        ],
        compiler_params=pltpu.CompilerParams(use_tc_tiling_on_sc=True,
                                             needs_layout_passes=False,
                                             disable_bounds_checks=True),
    )
    def fmt(tab_hbm, out_hbm, in0, in1, tr0, tr1, isem, osem):
        wid = lax.axis_index("s") * num_cores + lax.axis_index("c")
        start = upw * wid
        cnt = jnp.clip(full_tiles - start, 0, upw)
        invs = (in0, in1)
        trvs = (tr0, tr1)

        def c0_of(u):
            return pl.multiple_of((start + u) * 128, 128)

        def in_copy(u, p):
            return pltpu.make_async_copy(
                tab_hbm.at[:, pl.ds(c0_of(u), 128)],
                invs[p].at[:, pl.ds(0, 128)], isem.at[p])

        def out_copy(u, p):
            return pltpu.make_async_copy(
                trvs[p],
                out_hbm.at[pl.ds(pl.multiple_of(c0_of(u) * d, 8), 128 * d)],
                osem.at[p])

        iot = lax.iota(jnp.int32, _LANES)

        def transpose(inv, trv):
            @plsc.parallel_loop(0, 128, unroll=8)
            def _row_block(r):
                rvec = jnp.full((_LANES,), r, jnp.int32)
                for k in range(d // _LANES):
                    vals = plsc.load_gather(inv, [iot + k * _LANES, rvec])
                    trv[pl.ds(pl.multiple_of(r * d, 8) + k * _LANES,
                              _LANES)] = vals

        in_copy(0, 0).start()

        def group(g, carry):
            for p in range(2):
                u = g * 2 + p

                @pl.when(u < cnt)
                def _do():
                    in_copy(u, p).wait()

                    @pl.when(u + 1 < cnt)
                    def _prefetch():
                        in_copy(u + 1, 1 - p).start()

                    @pl.when(u >= 2)
                    def _free_buf():
                        out_copy(u - 2, p).wait()

                    transpose(invs[p], trvs[p])
                    out_copy(u, p).start()

            return carry

        lax.fori_loop(0, (upw + 1) // 2, group, 0)

        for p in range(2):
            @pl.when(((cnt - 2) % 2 == p) & (cnt >= 2))
            def _drain_m2():
                out_copy(cnt - 2, p).wait()

            @pl.when(((cnt - 1) % 2 == p) & (cnt >= 1))
            def _drain_m1():
                out_copy(cnt - 1, p).wait()

    return fmt(tab_t)


def _emb_gather(x2d, tab, out_shape, nb, num_cores, num_workers):
    b, l = x2d.shape
    d = tab.shape[1]
    b_per_w = b // num_workers
    n_chunks = b_per_w // nb
    n_groups = n_chunks // 2

    mesh = plsc.VectorSubcoreMesh(core_axis_name="c", subcore_axis_name="s")

    @functools.partial(
        pl.kernel,
        mesh=mesh,
        out_type=jax.ShapeDtypeStruct(out_shape, jnp.float32),
        scratch_types=[
            pltpu.VMEM((nb, l), jnp.int32),
            pltpu.VMEM((nb, l), jnp.int32),
            pltpu.VMEM((nb, l, d), jnp.float32),
            pltpu.VMEM((nb, l, d), jnp.float32),
            pltpu.SemaphoreType.DMA((2,)),
            pltpu.SemaphoreType.DMA((2,)),
        ],
        compiler_params=pltpu.CompilerParams(use_tc_tiling_on_sc=False),
    )
    def emb(x_hbm, tab_hbm, out_hbm, idx0, idx1, rows0, rows1, gsem, wsem):
        wid = lax.axis_index("s") * num_cores + lax.axis_index("c")
        base = wid * b_per_w
        idxs = (idx0, idx1)
        rows = (rows0, rows1)

        def start_gather(ci, p):
            pltpu.sync_copy(x_hbm.at[pl.ds(base + ci * nb, nb)], idxs[p])
            for j in range(nb):
                pltpu.async_copy(tab_hbm.at[idxs[p].at[j]], rows[p].at[j],
                                 gsem.at[p])

        def wait_gather(ci, p):
            for j in range(nb):
                pltpu.make_async_copy(tab_hbm.at[idxs[p].at[j]],
                                      rows[p].at[j], gsem.at[p]).wait()

        def wb_copy(ci, p):
            return pltpu.make_async_copy(
                rows[p], out_hbm.at[pl.ds(base + ci * nb, nb)], wsem.at[p])

        for p in range(2):
            start_gather(p, p)

        def group(g, carry):
            for p in range(2):
                ci = g * 2 + p
                wait_gather(ci, p)
                wb_copy(ci, p).start()
                wb_copy(ci, p).wait()
                start_gather(ci + 2, p)
            return carry

        lax.fori_loop(0, n_groups - 1, group, 0)

        for p in range(2):
            ci = (n_groups - 1) * 2 + p
            wait_gather(ci, p)
            wb_copy(ci, p).start()
        for p in range(2):
            ci = (n_groups - 1) * 2 + p
            wb_copy(ci, p).wait()

    return emb(x2d, tab)


def kernel(x, table):
    b, l = x.shape
    v, d = table.shape

    info = plsc.get_sparse_core_info()
    nw = info.num_cores * info.num_subcores

    tab_flat = _format_table(table.T, info.num_cores, nw)
    full_rows = (v // 128) * 128
    if full_rows < v:
        tail = table[full_rows:].reshape(-1)
        tab_flat = jax.lax.dynamic_update_slice(tab_flat, tail,
                                                (full_rows * d,))
    tab = tab_flat.reshape(v, d)

    x2d = x.astype(jnp.int32)
    return _emb_gather(x2d, tab, (b, l, d), 16, info.num_cores, nw)


# speedup vs baseline: 1.1601x; 1.1601x over previous
import functools

import jax
import jax.numpy as jnp
from jax import lax
from jax.experimental import pallas as pl
from jax.experimental.pallas import tpu as pltpu
from jax.experimental.pallas import tpu_sc as plsc

_NBUF = 2


def _emb_gather(x_flat, table, n_per_w, chunk, num_cores):
    n = x_flat.shape[0]
    d = table.shape[1]
    n_chunks = n_per_w // chunk
    n_groups = n_chunks // _NBUF

    mesh = plsc.VectorSubcoreMesh(core_axis_name="c", subcore_axis_name="s")

    @functools.partial(
        pl.kernel,
        mesh=mesh,
        out_type=jax.ShapeDtypeStruct((n, d), jnp.float32),
        scratch_types=[
            pltpu.VMEM((n_per_w,), jnp.int32),
            pltpu.VMEM((_NBUF, chunk, d), jnp.float32),
            pltpu.SemaphoreType.DMA((_NBUF,)),
            pltpu.SemaphoreType.DMA((_NBUF,)),
        ],
        compiler_params=pltpu.CompilerParams(use_tc_tiling_on_sc=False),
    )
    def emb(idx_hbm, table_hbm, out_hbm, idx_v, rows_v, gsem, wsem):
        wid = lax.axis_index("s") * num_cores + lax.axis_index("c")
        base = wid * n_per_w

        pltpu.sync_copy(idx_hbm.at[pl.ds(base, n_per_w)], idx_v)

        def idx_slice(ci):
            return idx_v.at[pl.ds(pl.multiple_of(ci * chunk, 8), chunk)]

        def start_gather(ci, b):
            pltpu.async_copy(table_hbm.at[idx_slice(ci)], rows_v.at[b],
                             gsem.at[b])

        def wait_gather(ci, b):
            pltpu.make_async_copy(table_hbm.at[idx_slice(ci)], rows_v.at[b],
                                  gsem.at[b]).wait()

        def out_slice(ci):
            return out_hbm.at[pl.ds(base + ci * chunk, chunk)]

        def start_wb(ci, b):
            pltpu.async_copy(rows_v.at[b], out_slice(ci), wsem.at[b])

        def wait_wb(ci, b):
            pltpu.make_async_copy(rows_v.at[b], out_slice(ci),
                                  wsem.at[b]).wait()

        for b in range(_NBUF):
            start_gather(b, b)

        def body(g, carry):
            for b in range(_NBUF):
                i = g * _NBUF + b
                wait_gather(i, b)
                start_wb(i, b)
                wait_wb(i, b)
                start_gather(i + _NBUF, b)
            return carry

        lax.fori_loop(0, n_groups - 1, body, 0)

        for b in range(_NBUF):
            i = (n_groups - 1) * _NBUF + b
            wait_gather(i, b)
            start_wb(i, b)
        for b in range(_NBUF):
            i = (n_groups - 1) * _NBUF + b
            wait_wb(i, b)

    return emb(x_flat, table)


def kernel(x, table):
    b, l = x.shape
    d = table.shape[1]
    n = b * l

    info = plsc.get_sparse_core_info()
    nw = info.num_cores * info.num_subcores
    n_per_w = n // nw
    chunk = 640

    x_flat = x.reshape(-1).astype(jnp.int32)
    out = _emb_gather(x_flat, table, n_per_w, chunk, info.num_cores)
    return out.reshape(b, l, d)
